# baseline (speedup 1.0000x reference)
import functools

import jax
import jax.numpy as jnp
from jax import lax
from jax.experimental import pallas as pl
from jax.experimental.pallas import tpu as pltpu
from jax.experimental.pallas import tpu_sc as plsc

BATCH = 16384
SEQ = 128
VOCAB = 1024
EMB = 32

NC = 2
NS = 16
NW = NC * NS
LANES = 16

SG = 2
BG = NW // SG
SPS = SEQ // SG
RPT = BATCH // BG
CH = 128
NCH = RPT // CH


def _prep_body(wr_ref, embt_ref, scores_ref, rowsum_ref):
    wr = wr_ref[...]
    embt = embt_ref[...]
    scores_ref[...] = jnp.dot(wr, embt, preferred_element_type=jnp.float32)
    rowsum_ref[...] = jnp.sum(embt, axis=0, keepdims=True)


def _epilogue_body(part_ref, hpart_ref, b_ref, out_ref):
    acc = jnp.sum(part_ref[...], axis=0, keepdims=True)
    gsum = jnp.sum(hpart_ref[...])
    out_ref[...] = jnp.maximum(acc + b_ref[...], 0.0) + gsum


def _sc_body(x_hbm, scores_hbm, rowsum_hbm, part_hbm, hpart_hbm,
             xc, table, rsv, accbuf, hbuf, semx0, semx1, sem1, sem2):
    cid = lax.axis_index("c")
    sid = lax.axis_index("s")
    wid = sid * NC + cid
    bg = wid // SG
    sg = wid % SG
    colbase = sg * SPS
    row0 = bg * RPT

    cp1 = pltpu.async_copy(scores_hbm.at[pl.ds(sg * SPS, SPS), :], table, sem1)
    cp2 = pltpu.async_copy(rowsum_hbm, rsv, sem2)
    pltpu.async_copy(x_hbm.at[pl.ds(row0, CH), :], xc.at[0], semx0)
    cp1.wait()
    cp2.wait()

    lanes = lax.iota(jnp.int32, LANES)

    def make_inner(buf):
        xck = xc.at[buf]

        def inner(base, h):
            @plsc.parallel_loop(0, CH // LANES, unroll=4, carry=h)
            def loop(c, h):
                rows = lanes + c * LANES
                acc = jnp.zeros((LANES,), jnp.float32)
                for s in range(SPS):
                    scol = jnp.full((LANES,), s, jnp.int32)
                    xv = plsc.load_gather(xck, [rows, scol + colbase])
                    sv = plsc.load_gather(table, [scol, xv])
                    rv = plsc.load_gather(rsv, [xv])
                    acc = acc + sv
                    h = h + rv
                accbuf[pl.ds(base + c * LANES, LANES)] = acc
                return h

            return loop

        return inner

    inner0 = make_inner(0)
    inner1 = make_inner(1)

    def xwait(buf):
        sem = semx0 if buf == 0 else semx1
        pltpu.make_async_copy(
            x_hbm.at[pl.ds(row0, CH), :], xc.at[buf], sem).wait()

    def outer(kp, h):
        c0 = kp * 2
        c1 = c0 + 1
        pltpu.async_copy(
            x_hbm.at[pl.ds(row0 + c1 * CH, CH), :], xc.at[1], semx1)
        xwait(0)
        h = inner0(c0 * CH, h)
        nxt = jnp.minimum(c1 + 1, NCH - 1)
        pltpu.async_copy(
            x_hbm.at[pl.ds(row0 + nxt * CH, CH), :], xc.at[0], semx0)
        xwait(1)
        h = inner1(c1 * CH, h)
        return h

    h = lax.fori_loop(0, NCH // 2, outer, jnp.zeros((LANES,), jnp.float32))
    xwait(0)

    hbuf[...] = h

    pltpu.sync_copy(accbuf, part_hbm.at[pl.ds((sg * BATCH) + bg * RPT, RPT)])
    pltpu.sync_copy(hbuf, hpart_hbm.at[pl.ds(wid * LANES, LANES)])


_sc_gather = functools.partial(
    pl.kernel,
    out_type=(
        jax.ShapeDtypeStruct((SG * BATCH,), jnp.float32),
        jax.ShapeDtypeStruct((NW * LANES,), jnp.float32),
    ),
    mesh=plsc.VectorSubcoreMesh(core_axis_name="c", subcore_axis_name="s"),
    compiler_params=pltpu.CompilerParams(needs_layout_passes=False),
    scratch_types=[
        pltpu.VMEM((2, CH, SEQ), jnp.int32),
        pltpu.VMEM((SPS, VOCAB), jnp.float32),
        pltpu.VMEM((VOCAB,), jnp.float32),
        pltpu.VMEM((RPT,), jnp.float32),
        pltpu.VMEM((LANES,), jnp.float32),
        pltpu.SemaphoreType.DMA,
        pltpu.SemaphoreType.DMA,
        pltpu.SemaphoreType.DMA,
        pltpu.SemaphoreType.DMA,
    ],
)(_sc_body)


def kernel(x, emb, W, b):
    wr = W.reshape(SEQ, EMB)
    embt = emb.T
    scores, rowsum = pl.pallas_call(
        _prep_body,
        out_shape=(
            jax.ShapeDtypeStruct((SEQ, VOCAB), jnp.float32),
            jax.ShapeDtypeStruct((1, VOCAB), jnp.float32),
        ),
    )(wr, embt)

    part, hpart = _sc_gather(x, scores, rowsum.reshape(VOCAB))

    out = pl.pallas_call(
        _epilogue_body,
        out_shape=jax.ShapeDtypeStruct((1, BATCH), jnp.float32),
    )(part.reshape(SG, BATCH), hpart.reshape(1, NW * LANES), b.reshape(1, 1))
    return out.reshape(BATCH, 1)


# speedup vs baseline: 1.5576x; 1.5576x over previous
import functools

import jax
import jax.numpy as jnp
from jax import lax
from jax.experimental import pallas as pl
from jax.experimental.pallas import tpu as pltpu
from jax.experimental.pallas import tpu_sc as plsc

BATCH = 16384
SEQ = 128
VOCAB = 1024
EMB = 32

NC = 2
NS = 16
NW = NC * NS
LANES = 16

SG = 2
BG = NW // SG
SPS = SEQ // SG
RPT = BATCH // BG
CH = 128
NCH = RPT // CH


def _prep_body(wr_ref, embt_ref, emb_ref, scores_ref, rep_ref):
    wr = wr_ref[...]
    embt = embt_ref[...]
    scores_ref[...] = jnp.dot(wr, embt, preferred_element_type=jnp.float32,
                              precision=lax.Precision.HIGHEST)
    ones = jnp.ones((EMB, LANES), jnp.float32)
    rep_ref[...] = jnp.dot(emb_ref[...], ones,
                           preferred_element_type=jnp.float32,
                           precision=lax.Precision.HIGHEST)


def _epilogue_body(part_ref, hpart_ref, b_ref, out_ref):
    acc = jnp.sum(part_ref[...], axis=0, keepdims=True)
    gsum = jnp.sum(hpart_ref[...])
    out_ref[...] = jnp.maximum(acc + b_ref[...], 0.0) + gsum


def _sc_body(x_hbm, scores_hbm, rep_hbm, part_hbm, hpart_hbm,
             xc, table, rsv, accbuf, hbuf, semx0, semx1, sem1, sem2):
    cid = lax.axis_index("c")
    sid = lax.axis_index("s")
    wid = sid * NC + cid
    bg = wid // SG
    sg = wid % SG
    colbase = sg * SPS
    row0 = bg * RPT

    cp1 = pltpu.async_copy(scores_hbm.at[pl.ds(sg * SPS, SPS), :], table, sem1)
    cp2 = pltpu.async_copy(rep_hbm, rsv, sem2)
    pltpu.async_copy(x_hbm.at[pl.ds(row0, CH), :], xc.at[0], semx0)
    cp1.wait()
    cp2.wait()

    lanes = lax.iota(jnp.int32, LANES)

    def make_inner(buf):
        xck = xc.at[buf]

        def inner(base, h):
            @plsc.parallel_loop(0, CH // LANES, unroll=2, carry=h)
            def loop(c, h):
                rows = lanes + c * LANES
                acc = jnp.zeros((LANES,), jnp.float32)
                for d in range(SPS):
                    svec = (jnp.full((LANES,), d, jnp.int32) + lanes) & (
                        SPS - 1)
                    xv = plsc.load_gather(xck, [rows, svec + colbase])
                    sv = plsc.load_gather(table, [svec, xv])
                    rv = plsc.load_gather(rsv, [xv * LANES + lanes])
                    acc = acc + sv
                    h = h + rv
                accbuf[pl.ds(base + c * LANES, LANES)] = acc
                return h

            return loop

        return inner

    inner0 = make_inner(0)
    inner1 = make_inner(1)

    def xwait(buf):
        sem = semx0 if buf == 0 else semx1
        pltpu.make_async_copy(
            x_hbm.at[pl.ds(row0, CH), :], xc.at[buf], sem).wait()

    def outer(kp, h):
        c0 = kp * 2
        c1 = c0 + 1
        pltpu.async_copy(
            x_hbm.at[pl.ds(row0 + c1 * CH, CH), :], xc.at[1], semx1)
        xwait(0)
        h = inner0(c0 * CH, h)
        nxt = jnp.minimum(c1 + 1, NCH - 1)
        pltpu.async_copy(
            x_hbm.at[pl.ds(row0 + nxt * CH, CH), :], xc.at[0], semx0)
        xwait(1)
        h = inner1(c1 * CH, h)
        return h

    h = lax.fori_loop(0, NCH // 2, outer, jnp.zeros((LANES,), jnp.float32))
    xwait(0)

    hbuf[...] = h

    pltpu.sync_copy(accbuf, part_hbm.at[pl.ds((sg * BATCH) + bg * RPT, RPT)])
    pltpu.sync_copy(hbuf, hpart_hbm.at[pl.ds(wid * LANES, LANES)])


_sc_gather = functools.partial(
    pl.kernel,
    out_type=(
        jax.ShapeDtypeStruct((SG * BATCH,), jnp.float32),
        jax.ShapeDtypeStruct((NW * LANES,), jnp.float32),
    ),
    mesh=plsc.VectorSubcoreMesh(core_axis_name="c", subcore_axis_name="s"),
    compiler_params=pltpu.CompilerParams(needs_layout_passes=False),
    scratch_types=[
        pltpu.VMEM((2, CH, SEQ), jnp.int32),
        pltpu.VMEM((SPS, VOCAB), jnp.float32),
        pltpu.VMEM((VOCAB * LANES,), jnp.float32),
        pltpu.VMEM((RPT,), jnp.float32),
        pltpu.VMEM((LANES,), jnp.float32),
        pltpu.SemaphoreType.DMA,
        pltpu.SemaphoreType.DMA,
        pltpu.SemaphoreType.DMA,
        pltpu.SemaphoreType.DMA,
    ],
)(_sc_body)


def kernel(x, emb, W, b):
    wr = W.reshape(SEQ, EMB)
    embt = emb.T
    scores, rep = pl.pallas_call(
        _prep_body,
        out_shape=(
            jax.ShapeDtypeStruct((SEQ, VOCAB), jnp.float32),
            jax.ShapeDtypeStruct((VOCAB, LANES), jnp.float32),
        ),
    )(wr, embt, emb)

    part, hpart = _sc_gather(x, scores, rep.reshape(VOCAB * LANES))

    out = pl.pallas_call(
        _epilogue_body,
        out_shape=jax.ShapeDtypeStruct((1, BATCH), jnp.float32),
    )(part.reshape(SG, BATCH), hpart.reshape(1, NW * LANES), b.reshape(1, 1))
    return out.reshape(BATCH, 1)


# speedup vs baseline: 1.6862x; 1.0826x over previous
import functools

import jax
import jax.numpy as jnp
from jax import lax
from jax.experimental import pallas as pl
from jax.experimental.pallas import tpu as pltpu
from jax.experimental.pallas import tpu_sc as plsc

BATCH = 16384
SEQ = 128
VOCAB = 1024
EMB = 32

NC = 2
NS = 16
NW = NC * NS
LANES = 16

SG = 2
BG = NW // SG
SPS = SEQ // SG
RPT = BATCH // BG
CH = 128
NCH = RPT // CH


def _prep_body(wr_ref, emb_ref, emb2_ref, scores_ref, rep_ref):
    wr = wr_ref[...]
    emb = emb_ref[...]
    scores_ref[...] = lax.dot_general(
        wr, emb, (((1,), (1,)), ((), ())),
        preferred_element_type=jnp.float32,
        precision=lax.Precision.HIGHEST)
    kk = lax.broadcasted_iota(jnp.int32, (8 * EMB, SEQ), 0) // EMB
    cc = lax.broadcasted_iota(jnp.int32, (8 * EMB, SEQ), 1) // LANES
    sel = jnp.where(kk == cc, 1.0, 0.0).astype(jnp.float32)
    rep_ref[...] = jnp.dot(emb2_ref[...], sel,
                           preferred_element_type=jnp.float32,
                           precision=lax.Precision.HIGHEST)


def _epilogue_body(part_ref, hpart_ref, b_ref, out_ref):
    part = part_ref[...]
    acc = part[:, :BATCH] + part[:, BATCH:]
    gsum = jnp.sum(hpart_ref[...])
    out_ref[...] = jnp.maximum(acc + b_ref[...], 0.0) + gsum


def _sc_body(x_hbm, scores_hbm, rep_hbm, part_hbm, hpart_hbm,
             xc, table, rsv, accbuf, hbuf, semx0, semx1, sem1, sem2):
    cid = lax.axis_index("c")
    sid = lax.axis_index("s")
    wid = sid * NC + cid
    bg = wid // SG
    sg = wid % SG
    colbase = sg * SPS
    row0 = bg * RPT

    cp1 = pltpu.async_copy(scores_hbm.at[pl.ds(sg * SPS, SPS), :], table, sem1)
    cp2 = pltpu.async_copy(rep_hbm, rsv, sem2)
    pltpu.async_copy(x_hbm.at[pl.ds(row0, CH), :], xc.at[0], semx0)
    cp1.wait()
    cp2.wait()

    lanes = lax.iota(jnp.int32, LANES)

    def make_inner(buf):
        xck = xc.at[buf]

        def inner(base, h):
            @plsc.parallel_loop(0, CH // LANES, unroll=2, carry=h)
            def loop(c, h):
                rows = lanes + c * LANES
                acc = jnp.zeros((LANES,), jnp.float32)
                for d in range(SPS):
                    svec = (jnp.full((LANES,), d, jnp.int32) + lanes) & (
                        SPS - 1)
                    xv = plsc.load_gather(xck, [rows, svec + colbase])
                    sv = plsc.load_gather(table, [svec, xv])
                    rvi = xv * LANES + lanes
                    rv = plsc.load_gather(rsv, [rvi >> 7, rvi & 127])
                    acc = acc + sv
                    h = h + rv
                accbuf[pl.ds(base + c * LANES, LANES)] = acc
                return h

            return loop

        return inner

    inner0 = make_inner(0)
    inner1 = make_inner(1)

    def xwait(buf):
        sem = semx0 if buf == 0 else semx1
        pltpu.make_async_copy(
            x_hbm.at[pl.ds(row0, CH), :], xc.at[buf], sem).wait()

    def outer(kp, h):
        c0 = kp * 2
        c1 = c0 + 1
        pltpu.async_copy(
            x_hbm.at[pl.ds(row0 + c1 * CH, CH), :], xc.at[1], semx1)
        xwait(0)
        h = inner0(c0 * CH, h)
        nxt = jnp.minimum(c1 + 1, NCH - 1)
        pltpu.async_copy(
            x_hbm.at[pl.ds(row0 + nxt * CH, CH), :], xc.at[0], semx0)
        xwait(1)
        h = inner1(c1 * CH, h)
        return h

    h = lax.fori_loop(0, NCH // 2, outer, jnp.zeros((LANES,), jnp.float32))
    xwait(0)

    hbuf[...] = h

    pltpu.sync_copy(accbuf, part_hbm.at[pl.ds((sg * BATCH) + bg * RPT, RPT)])
    pltpu.sync_copy(hbuf, hpart_hbm.at[pl.ds(wid * LANES, LANES)])


_sc_gather = functools.partial(
    pl.kernel,
    out_type=(
        jax.ShapeDtypeStruct((SG * BATCH,), jnp.float32),
        jax.ShapeDtypeStruct((NW * LANES,), jnp.float32),
    ),
    mesh=plsc.VectorSubcoreMesh(core_axis_name="c", subcore_axis_name="s"),
    compiler_params=pltpu.CompilerParams(needs_layout_passes=False),
    scratch_types=[
        pltpu.VMEM((2, CH, SEQ), jnp.int32),
        pltpu.VMEM((SPS, VOCAB), jnp.float32),
        pltpu.VMEM((VOCAB * LANES // SEQ, SEQ), jnp.float32),
        pltpu.VMEM((RPT,), jnp.float32),
        pltpu.VMEM((LANES,), jnp.float32),
        pltpu.SemaphoreType.DMA,
        pltpu.SemaphoreType.DMA,
        pltpu.SemaphoreType.DMA,
        pltpu.SemaphoreType.DMA,
    ],
)(_sc_body)


def kernel(x, emb, W, b):
    wr = W.reshape(SEQ, EMB)
    emb2 = emb.reshape(SEQ, 8 * EMB)
    scores, rep = pl.pallas_call(
        _prep_body,
        out_shape=(
            jax.ShapeDtypeStruct((SEQ, VOCAB), jnp.float32),
            jax.ShapeDtypeStruct((VOCAB * LANES // SEQ, SEQ), jnp.float32),
        ),
    )(wr, emb, emb2)

    part, hpart = _sc_gather(x, scores, rep)

    out = pl.pallas_call(
        _epilogue_body,
        out_shape=jax.ShapeDtypeStruct((1, BATCH), jnp.float32),
    )(part.reshape(1, SG * BATCH), hpart.reshape(1, NW * LANES),
      b.reshape(1, 1))
    return out.reshape(BATCH, 1)
